# baseline (speedup 1.0000x reference)
import functools

import jax
import jax.numpy as jnp
from jax import lax
from jax.experimental import pallas as pl
from jax.experimental.pallas import tpu as pltpu
from jax.experimental.pallas import tpu_sc as plsc

TOPK = 50
LANES = 128
RB_A = 32
RB_B = 128
NEG = -3.0e38
BIGI = 2**30


def _chunksel_body(yt_ref, cid_ref):
    y = yt_ref[...]
    r, _, nchunk = y.shape
    cm = jnp.max(y, axis=1)
    citer = lax.broadcasted_iota(jnp.int32, (r, nchunk), 1)
    for t in range(TOPK):
        m = jnp.max(cm, axis=1, keepdims=True)
        pos = jnp.min(jnp.where(cm == m, citer, BIGI), axis=1, keepdims=True)
        cm = jnp.where(citer == pos, NEG, cm)
        cid_ref[:, t:t + 1] = pos


def _extract_body(ncols, cv_ref, gi_ref, idx_ref):
    cv = cv_ref[...]
    gi = gi_ref[...]
    cv = jnp.where(gi >= ncols, NEG, cv)
    for t in range(TOPK):
        m = jnp.max(cv, axis=1, keepdims=True)
        pos = jnp.min(jnp.where(cv == m, gi, BIGI), axis=1, keepdims=True)
        cv = jnp.where(gi == pos, NEG, cv)
        idx_ref[:, t:t + 1] = pos


def _sc_gather(table, idx_flat):
    info = plsc.get_sparse_core_info()
    nw = info.num_cores * info.num_subcores
    b = idx_flat.shape[0]
    d = table.shape[1]
    b_per_w = b // nw
    nsplit = 1
    while (b_per_w // nsplit) * d * 4 > 420_000 or b_per_w % nsplit:
        nsplit += 1
    b_sub = b_per_w // nsplit
    mesh = plsc.VectorSubcoreMesh(core_axis_name="c", subcore_axis_name="s")

    @functools.partial(
        pl.kernel, mesh=mesh,
        compiler_params=pltpu.CompilerParams(use_tc_tiling_on_sc=False),
        out_type=jax.ShapeDtypeStruct((b, d), jnp.float32),
        scratch_types=[
            pltpu.VMEM((b_sub,), jnp.int32),
            pltpu.VMEM((b_sub, d), jnp.float32),
            pltpu.SemaphoreType.DMA,
        ],
    )
    def k(table_hbm, idx_hbm, out_hbm, idx_v, rows_v, sem):
        wid = lax.axis_index("s") * info.num_cores + lax.axis_index("c")
        for j in range(nsplit):
            base = wid * b_per_w + j * b_sub
            pltpu.sync_copy(idx_hbm.at[pl.ds(base, b_sub)], idx_v)
            pltpu.async_copy(table_hbm.at[idx_v], rows_v, sem).wait()
            pltpu.sync_copy(rows_v, out_hbm.at[pl.ds(base, b_sub)])

    return k(table, idx_flat)


def kernel(tags, embed_table):
    nrows, ncols = tags.shape
    nchunk = (ncols + LANES - 1) // LANES
    pad = nchunk * LANES - ncols
    xp = jnp.pad(tags, ((0, 0), (0, pad)), constant_values=NEG)
    yt = xp.reshape(nrows, nchunk, LANES).transpose(0, 2, 1)

    cids = pl.pallas_call(
        _chunksel_body,
        grid=(nrows // RB_A,),
        in_specs=[pl.BlockSpec((RB_A, LANES, nchunk), lambda i: (i, 0, 0))],
        out_specs=pl.BlockSpec((RB_A, TOPK), lambda i: (i, 0)),
        out_shape=jax.ShapeDtypeStruct((nrows, TOPK), jnp.int32),
    )(yt)

    sub = 32
    chunk_rows = tags.reshape(nrows * ncols // sub, sub)
    base = (cids * (LANES // sub)
            + jnp.arange(nrows, dtype=jnp.int32)[:, None] * (ncols // sub))
    sub_idx = base[:, :, None] + jnp.arange(LANES // sub, dtype=jnp.int32)
    sub_idx = jnp.minimum(sub_idx, nrows * ncols // sub - 1)
    cand = _sc_gather(chunk_rows, sub_idx.reshape(-1))
    cv = cand.reshape(nrows, TOPK * LANES)
    gi = (cids[:, :, None] * LANES
          + jnp.arange(LANES, dtype=jnp.int32)).reshape(nrows, TOPK * LANES)

    idx = pl.pallas_call(
        functools.partial(_extract_body, ncols),
        grid=(nrows // RB_B,),
        in_specs=[pl.BlockSpec((RB_B, TOPK * LANES), lambda i: (i, 0)),
                  pl.BlockSpec((RB_B, TOPK * LANES), lambda i: (i, 0))],
        out_specs=pl.BlockSpec((RB_B, TOPK), lambda i: (i, 0)),
        out_shape=jax.ShapeDtypeStruct((nrows, TOPK), jnp.int32),
    )(cv, gi)

    rows = _sc_gather(embed_table, idx.reshape(-1))
    return rows.reshape(nrows, TOPK, embed_table.shape[1])


# speedup vs baseline: 1.0601x; 1.0601x over previous
import functools

import jax
import jax.numpy as jnp
from jax import lax
from jax.experimental import pallas as pl
from jax.experimental.pallas import tpu as pltpu
from jax.experimental.pallas import tpu_sc as plsc

TOPK = 50
LANES = 128
RB_A = 32
RB_B = 128
NEG = -3.0e38
BIGI = 2**30


def _chunksel_body(ncols, x_ref, cid_ref):
    x = x_ref[...]
    r = x.shape[0]
    nchunk = x.shape[1] // LANES
    col = lax.broadcasted_iota(jnp.int32, (r, nchunk * LANES), 1)
    x = jnp.where(col >= ncols, NEG, x)
    cm = jnp.max(x.reshape(r, nchunk, LANES), axis=2)
    citer = lax.broadcasted_iota(jnp.int32, (r, nchunk), 1)
    for t in range(TOPK):
        m = jnp.max(cm, axis=1, keepdims=True)
        pos = jnp.min(jnp.where(cm == m, citer, BIGI), axis=1, keepdims=True)
        cm = jnp.where(citer == pos, NEG, cm)
        cid_ref[:, t:t + 1] = pos


def _extract_body(ncols, cv_ref, gi_ref, idx_ref):
    cv = cv_ref[...]
    gi = gi_ref[...]
    cv = jnp.where(gi >= ncols, NEG, cv)
    for t in range(TOPK):
        m = jnp.max(cv, axis=1, keepdims=True)
        pos = jnp.min(jnp.where(cv == m, gi, BIGI), axis=1, keepdims=True)
        cv = jnp.where(gi == pos, NEG, cv)
        idx_ref[:, t:t + 1] = pos


def _sc_gather(table, idx_flat):
    info = plsc.get_sparse_core_info()
    nw = info.num_cores * info.num_subcores
    b = idx_flat.shape[0]
    d = table.shape[1]
    b_per_w = b // nw
    nsplit = 1
    while (b_per_w // nsplit) * d * 4 > 420_000 or b_per_w % nsplit:
        nsplit += 1
    b_sub = b_per_w // nsplit
    mesh = plsc.VectorSubcoreMesh(core_axis_name="c", subcore_axis_name="s")

    @functools.partial(
        pl.kernel, mesh=mesh,
        compiler_params=pltpu.CompilerParams(use_tc_tiling_on_sc=False),
        out_type=jax.ShapeDtypeStruct((b, d), jnp.float32),
        scratch_types=[
            pltpu.VMEM((b_sub,), jnp.int32),
            pltpu.VMEM((b_sub, d), jnp.float32),
            pltpu.SemaphoreType.DMA,
        ],
    )
    def k(table_hbm, idx_hbm, out_hbm, idx_v, rows_v, sem):
        wid = lax.axis_index("s") * info.num_cores + lax.axis_index("c")
        for j in range(nsplit):
            base = wid * b_per_w + j * b_sub
            pltpu.sync_copy(idx_hbm.at[pl.ds(base, b_sub)], idx_v)
            pltpu.async_copy(table_hbm.at[idx_v], rows_v, sem).wait()
            pltpu.sync_copy(rows_v, out_hbm.at[pl.ds(base, b_sub)])

    return k(table, idx_flat)


def kernel(tags, embed_table):
    nrows, ncols = tags.shape
    nchunk = (ncols + LANES - 1) // LANES

    cids = pl.pallas_call(
        functools.partial(_chunksel_body, ncols),
        grid=(nrows // RB_A,),
        in_specs=[pl.BlockSpec((RB_A, nchunk * LANES), lambda i: (i, 0))],
        out_specs=pl.BlockSpec((RB_A, TOPK), lambda i: (i, 0)),
        out_shape=jax.ShapeDtypeStruct((nrows, TOPK), jnp.int32),
    )(tags)

    sub = 32
    chunk_rows = tags.reshape(nrows * ncols // sub, sub)
    base = (cids * (LANES // sub)
            + jnp.arange(nrows, dtype=jnp.int32)[:, None] * (ncols // sub))
    sub_idx = base[:, :, None] + jnp.arange(LANES // sub, dtype=jnp.int32)
    sub_idx = jnp.minimum(sub_idx, nrows * ncols // sub - 1)
    cand = _sc_gather(chunk_rows, sub_idx.reshape(-1))
    cv = cand.reshape(nrows, TOPK * LANES)
    gi = (cids[:, :, None] * LANES
          + jnp.arange(LANES, dtype=jnp.int32)).reshape(nrows, TOPK * LANES)

    idx = pl.pallas_call(
        functools.partial(_extract_body, ncols),
        grid=(nrows // RB_B,),
        in_specs=[pl.BlockSpec((RB_B, TOPK * LANES), lambda i: (i, 0)),
                  pl.BlockSpec((RB_B, TOPK * LANES), lambda i: (i, 0))],
        out_specs=pl.BlockSpec((RB_B, TOPK), lambda i: (i, 0)),
        out_shape=jax.ShapeDtypeStruct((nrows, TOPK), jnp.int32),
    )(cv, gi)

    rows = _sc_gather(embed_table, idx.reshape(-1))
    return rows.reshape(nrows, TOPK, embed_table.shape[1])


# speedup vs baseline: 1.0747x; 1.0137x over previous
import functools

import jax
import jax.numpy as jnp
from jax import lax
from jax.experimental import pallas as pl
from jax.experimental.pallas import tpu as pltpu
from jax.experimental.pallas import tpu_sc as plsc

TOPK = 50
LANES = 128
RB_A = 16
RB_B = 128
NEG = -3.0e38
BIGI = 2**30


def _chunksel_body(ncols, x_ref, cid_ref, chunks_ref):
    x = x_ref[...]
    r = x.shape[0]
    nchunk = x.shape[1] // LANES
    col = lax.broadcasted_iota(jnp.int32, (r, nchunk * LANES), 1)
    x = jnp.where(col >= ncols, NEG, x)
    chunks_ref[...] = x.reshape(r * nchunk, LANES)
    cm = jnp.max(x.reshape(r, nchunk, LANES), axis=2)
    citer = lax.broadcasted_iota(jnp.int32, (r, nchunk), 1)
    for t in range(TOPK):
        m = jnp.max(cm, axis=1, keepdims=True)
        pos = jnp.min(jnp.where(cm == m, citer, BIGI), axis=1, keepdims=True)
        cm = jnp.where(citer == pos, NEG, cm)
        cid_ref[:, t:t + 1] = pos


def _extract_body(ncols, cv_ref, gi_ref, idx_ref):
    cv = cv_ref[...]
    gi = gi_ref[...]
    cv = jnp.where(gi >= ncols, NEG, cv)
    for t in range(TOPK):
        m = jnp.max(cv, axis=1, keepdims=True)
        pos = jnp.min(jnp.where(cv == m, gi, BIGI), axis=1, keepdims=True)
        cv = jnp.where(gi == pos, NEG, cv)
        idx_ref[:, t:t + 1] = pos


def _sc_gather(table, idx_flat, tc_tiling=False):
    info = plsc.get_sparse_core_info()
    nw = info.num_cores * info.num_subcores
    b = idx_flat.shape[0]
    d = table.shape[1]
    b_per_w = b // nw
    nsplit = 1
    while (b_per_w // nsplit) * d * 4 > 420_000 or b_per_w % nsplit:
        nsplit += 1
    b_sub = b_per_w // nsplit
    mesh = plsc.VectorSubcoreMesh(core_axis_name="c", subcore_axis_name="s")

    @functools.partial(
        pl.kernel, mesh=mesh,
        compiler_params=pltpu.CompilerParams(use_tc_tiling_on_sc=tc_tiling),
        out_type=jax.ShapeDtypeStruct((b, d), jnp.float32),
        scratch_types=[
            pltpu.VMEM((b_sub,), jnp.int32),
            pltpu.VMEM((b_sub, d), jnp.float32),
            pltpu.SemaphoreType.DMA,
        ],
    )
    def k(table_hbm, idx_hbm, out_hbm, idx_v, rows_v, sem):
        wid = lax.axis_index("s") * info.num_cores + lax.axis_index("c")
        for j in range(nsplit):
            base = wid * b_per_w + j * b_sub
            pltpu.sync_copy(idx_hbm.at[pl.ds(base, b_sub)], idx_v)
            pltpu.async_copy(table_hbm.at[idx_v], rows_v, sem).wait()
            pltpu.sync_copy(rows_v, out_hbm.at[pl.ds(base, b_sub)])

    return k(table, idx_flat)


def kernel(tags, embed_table):
    nrows, ncols = tags.shape
    nchunk = (ncols + LANES - 1) // LANES

    cids, chunk_rows = pl.pallas_call(
        functools.partial(_chunksel_body, ncols),
        grid=(nrows // RB_A,),
        in_specs=[pl.BlockSpec((RB_A, nchunk * LANES), lambda i: (i, 0))],
        out_specs=[pl.BlockSpec((RB_A, TOPK), lambda i: (i, 0)),
                   pl.BlockSpec((RB_A * nchunk, LANES), lambda i: (i, 0))],
        out_shape=[jax.ShapeDtypeStruct((nrows, TOPK), jnp.int32),
                   jax.ShapeDtypeStruct((nrows * nchunk, LANES),
                                        jnp.float32)],
    )(tags)

    row_ids = (cids
               + jnp.arange(nrows, dtype=jnp.int32)[:, None] * nchunk)
    cand = _sc_gather(chunk_rows, row_ids.reshape(-1),
                      tc_tiling=True)
    cv = cand.reshape(nrows, TOPK * LANES)
    gi = (cids[:, :, None] * LANES
          + jnp.arange(LANES, dtype=jnp.int32)).reshape(nrows, TOPK * LANES)

    idx = pl.pallas_call(
        functools.partial(_extract_body, ncols),
        grid=(nrows // RB_B,),
        in_specs=[pl.BlockSpec((RB_B, TOPK * LANES), lambda i: (i, 0)),
                  pl.BlockSpec((RB_B, TOPK * LANES), lambda i: (i, 0))],
        out_specs=pl.BlockSpec((RB_B, TOPK), lambda i: (i, 0)),
        out_shape=jax.ShapeDtypeStruct((nrows, TOPK), jnp.int32),
    )(cv, gi)

    rows = _sc_gather(embed_table, idx.reshape(-1))
    return rows.reshape(nrows, TOPK, embed_table.shape[1])
